# baseline (speedup 1.0000x reference)
import jax
import jax.numpy as jnp
import numpy as np
from jax.experimental import pallas as pl
from jax.experimental.pallas import tpu as pltpu

_LAMBDA_MU = 0.1
_LAMBDA_L = 0.1
_N_SUBJECTS = 16
_LOG2PI = float(np.log(2.0 * np.pi))


def _body(sid_ref, ids_ref, xgT_ref, mu_subj_ref,
          mu_pop_ref, Ls_ref, Lp_ref, out_ref, ahat_scr):
    B, T, D, K, P = 16, 512, 32, 8, 16
    DA = D + 1
    f32 = jnp.float32

    L = Ls_ref[...]
    ii = jax.lax.broadcasted_iota(jnp.int32, (D, D), 0)
    jj = jax.lax.broadcasted_iota(jnp.int32, (D, D), 1)
    eye = (ii == jj).astype(f32)
    strict_lo = (ii > jj).astype(f32)

    d = jnp.sum(L * eye, axis=1)
    rinv = 1.0 / d
    N = -(L * strict_lo) * rinv[:, :, None]
    bmm = lambda a, b, dn: jax.lax.dot_general(
        a, b, (dn, ((0,), (0,))), preferred_element_type=f32)
    X = eye[None] + N
    Npow = N
    for _ in range(4):
        Npow = bmm(Npow, Npow, ((2,), (1,)))
        X = X + bmm(X, Npow, ((2,), (1,)))
    Linv = X * rinv[:, None, :]

    A = bmm(Linv, Linv, ((1,), (1,)))
    mu = mu_subj_ref[...]
    bvec = jnp.sum(A * mu[:, None, :], axis=2)
    cval = jnp.sum(mu * bvec, axis=1, keepdims=True)
    logdet = jnp.sum(jnp.log(d), axis=1, keepdims=True)
    Cval = (-0.5 * D * _LOG2PI) - logdet

    ahat_scr[:, 0:D, 0:D] = A
    ahat_scr[:, D:DA, 0:D] = -bvec[:, None, :]
    ahat_scr[:, 0:D, D:DA] = -bvec[:, :, None]
    ahat_scr[:, D:DA, D:DA] = (cval - 2.0 * Cval)[:, :, None]

    DG = DA + K
    eyeg = (jax.lax.broadcasted_iota(jnp.int32, (DG, DG), 0) ==
            jax.lax.broadcasted_iota(jnp.int32, (DG, DG), 1)).astype(f32)
    acc = jnp.zeros((), dtype=f32)
    for b in range(B):
        s = sid_ref[b]
        xg = xgT_ref[b]
        XG = jnp.transpose(xg, (1, 0))
        xT = XG[0:DA]
        gT = XG[DA:DA + K]
        Wg = gT[:, None, :] * xT[None]
        S = bmm(Wg, jnp.broadcast_to(xg[None, :, 0:DA], (K, T, DA)),
                ((2,), (1,)))
        Ah = ahat_scr[pl.ds(s * K, K)]
        acc = acc + jnp.sum(Ah * S)
    nll = 0.5 * acc / float(B * T)

    ids_v = ids_ref[...]
    pio = jax.lax.broadcasted_iota(jnp.int32, (P, B), 0)
    pres = jnp.max((pio == ids_v).astype(f32), axis=1, keepdims=True)
    S_cnt = jnp.sum(pres)

    md = mu.reshape(P, K, D) - mu_pop_ref[...][None]
    msq = jnp.sum(jnp.sum(md * md, axis=2), axis=1, keepdims=True)
    mu_reg = jnp.sum(pres * msq)

    Ld = L.reshape(P, K, D, D) - Lp_ref[...][None]
    lsq = jnp.sum(jnp.sum(jnp.sum(Ld * Ld, axis=3), axis=2), axis=1,
                  keepdims=True)
    L_reg = jnp.sum(pres * lsq)

    reg = (S_cnt / _N_SUBJECTS) * (
        _LAMBDA_MU / 2.0 * mu_reg + _LAMBDA_L / 2.0 * L_reg)
    out_ref[0, 0] = nll + reg


def kernel(x, mu_pop, L_pop, mu_subj, L_subj, gamma, subject_ids):
    B, T, D = x.shape
    K = mu_pop.shape[0]
    P = mu_subj.shape[0]
    f32 = jnp.float32

    sid = subject_ids.astype(jnp.int32)
    xgT = jnp.concatenate([x, jnp.ones((B, T, 1), f32), gamma], axis=2)
    Ls = L_subj.reshape(P * K, D, D)
    mu_s = mu_subj.reshape(P * K, D)
    ids2 = sid.reshape(1, B)

    out = pl.pallas_call(
        _body,
        out_shape=jax.ShapeDtypeStruct((1, 1), f32),
        in_specs=[
            pl.BlockSpec(memory_space=pltpu.SMEM),
            pl.BlockSpec(memory_space=pltpu.VMEM),
            pl.BlockSpec(memory_space=pltpu.VMEM),
            pl.BlockSpec(memory_space=pltpu.VMEM),
            pl.BlockSpec(memory_space=pltpu.VMEM),
            pl.BlockSpec(memory_space=pltpu.VMEM),
            pl.BlockSpec(memory_space=pltpu.VMEM),
        ],
        out_specs=pl.BlockSpec(memory_space=pltpu.SMEM),
        scratch_shapes=[
            pltpu.VMEM((P * K, D + 1, D + 1), f32),
        ],
    )(sid, ids2, xgT, mu_s, mu_pop, Ls, L_pop)
    return out[0, 0]


# speedup vs baseline: 1.0293x; 1.0293x over previous
import jax
import jax.numpy as jnp
import numpy as np
from jax.experimental import pallas as pl
from jax.experimental.pallas import tpu as pltpu

_LAMBDA_MU = 0.1
_LAMBDA_L = 0.1
_N_SUBJECTS = 16
_LOG2PI = float(np.log(2.0 * np.pi))


def _body(sid_ref, ids_ref, xgT_ref, mu_subj_ref,
          mu_pop_ref, Ls_ref, Lp_ref, out_ref, ahat_scr):
    B, T, D, K, P = 16, 512, 32, 8, 16
    DA = D + 1
    f32 = jnp.float32
    b = pl.program_id(0)

    bmm = lambda a, bb, dn: jax.lax.dot_general(
        a, bb, (dn, ((0,), (0,))), preferred_element_type=f32)

    @pl.when(b == 0)
    def _init():
        L = Ls_ref[...]
        ii = jax.lax.broadcasted_iota(jnp.int32, (D, D), 0)
        jj = jax.lax.broadcasted_iota(jnp.int32, (D, D), 1)
        eye = (ii == jj).astype(f32)
        strict_lo = (ii > jj).astype(f32)

        d = jnp.sum(L * eye, axis=1)
        rinv = 1.0 / d
        N = -(L * strict_lo) * rinv[:, :, None]
        X = eye[None] + N
        Npow = N
        for _ in range(4):
            Npow = bmm(Npow, Npow, ((2,), (1,)))
            X = X + bmm(X, Npow, ((2,), (1,)))
        Linv = X * rinv[:, None, :]

        A = bmm(Linv, Linv, ((1,), (1,)))
        mu = mu_subj_ref[...]
        bvec = jnp.sum(A * mu[:, None, :], axis=2)
        cval = jnp.sum(mu * bvec, axis=1, keepdims=True)
        logdet = jnp.sum(jnp.log(d), axis=1, keepdims=True)
        Cval = (-0.5 * D * _LOG2PI) - logdet

        ahat_scr[:, 0:D, 0:D] = A
        ahat_scr[:, D:DA, 0:D] = -bvec[:, None, :]
        ahat_scr[:, 0:D, D:DA] = -bvec[:, :, None]
        ahat_scr[:, D:DA, D:DA] = (cval - 2.0 * Cval)[:, :, None]

        ids_v = ids_ref[...]
        pio = jax.lax.broadcasted_iota(jnp.int32, (P, B), 0)
        pres = jnp.max((pio == ids_v).astype(f32), axis=1, keepdims=True)
        S_cnt = jnp.sum(pres)

        md = mu.reshape(P, K, D) - mu_pop_ref[...][None]
        msq = jnp.sum(jnp.sum(md * md, axis=2), axis=1, keepdims=True)
        mu_reg = jnp.sum(pres * msq)

        Ld = L.reshape(P, K, D, D) - Lp_ref[...][None]
        lsq = jnp.sum(jnp.sum(jnp.sum(Ld * Ld, axis=3), axis=2), axis=1,
                      keepdims=True)
        L_reg = jnp.sum(pres * lsq)

        out_ref[0, 0] = (S_cnt / _N_SUBJECTS) * (
            _LAMBDA_MU / 2.0 * mu_reg + _LAMBDA_L / 2.0 * L_reg)

    s = sid_ref[b]
    XG = xgT_ref[0]
    xT = XG[0:DA]
    gT = XG[DA:DA + K]
    Wg = gT[:, None, :] * xT[None]
    S = bmm(Wg, jnp.broadcast_to(xT[None], (K, DA, T)), ((2,), (2,)))
    Ah = ahat_scr[pl.ds(s * K, K)]
    out_ref[0, 0] += 0.5 * jnp.sum(Ah * S) / float(B * T)


def kernel(x, mu_pop, L_pop, mu_subj, L_subj, gamma, subject_ids):
    B, T, D = x.shape
    K = mu_pop.shape[0]
    P = mu_subj.shape[0]
    f32 = jnp.float32

    sid = subject_ids.astype(jnp.int32)
    xg = jnp.concatenate([x, jnp.ones((B, T, 1), f32), gamma], axis=2)
    xgT = jnp.swapaxes(xg, 1, 2)
    Ls = L_subj.reshape(P * K, D, D)
    mu_s = mu_subj.reshape(P * K, D)
    ids2 = sid.reshape(1, B)
    DG = D + 1 + K

    grid_spec = pltpu.PrefetchScalarGridSpec(
        num_scalar_prefetch=1,
        grid=(B,),
        in_specs=[
            pl.BlockSpec((1, B), lambda b, sid_p: (0, 0)),
            pl.BlockSpec((1, DG, T), lambda b, sid_p: (b, 0, 0)),
            pl.BlockSpec((P * K, D), lambda b, sid_p: (0, 0)),
            pl.BlockSpec((K, D), lambda b, sid_p: (0, 0)),
            pl.BlockSpec((P * K, D, D), lambda b, sid_p: (0, 0, 0)),
            pl.BlockSpec((K, D, D), lambda b, sid_p: (0, 0, 0)),
        ],
        out_specs=pl.BlockSpec(
            (1, 1), lambda b, sid_p: (0, 0), memory_space=pltpu.SMEM),
        scratch_shapes=[
            pltpu.VMEM((P * K, D + 1, D + 1), f32),
        ],
    )
    out = pl.pallas_call(
        _body,
        grid_spec=grid_spec,
        out_shape=jax.ShapeDtypeStruct((1, 1), f32),
    )(sid, ids2, xgT, mu_s, mu_pop, Ls, L_pop)
    return out[0, 0]


# speedup vs baseline: 1.8008x; 1.7496x over previous
import jax
import jax.numpy as jnp
import numpy as np
from jax.experimental import pallas as pl
from jax.experimental.pallas import tpu as pltpu

_LAMBDA_MU = 0.1
_LAMBDA_L = 0.1
_N_SUBJECTS = 16
_LOG2PI = float(np.log(2.0 * np.pi))


def _body(sid_ref, ids_ref, xgT_ref, mu_subj_ref,
          mu_pop_ref, Ls_ref, Lp_ref, out_ref, ahat_scr, smom_scr):
    B, T, D, K, P = 16, 512, 32, 8, 16
    DA = D + 1
    f32 = jnp.float32

    L = Ls_ref[...]
    ii = jax.lax.broadcasted_iota(jnp.int32, (D, D), 0)
    jj = jax.lax.broadcasted_iota(jnp.int32, (D, D), 1)
    eye = (ii == jj).astype(f32)
    strict_lo = (ii > jj).astype(f32)

    d = jnp.sum(L * eye, axis=1)
    rinv = 1.0 / d
    N = -(L * strict_lo) * rinv[:, :, None]
    bmm = lambda a, b, dn: jax.lax.dot_general(
        a, b, (dn, ((0,), (0,))), preferred_element_type=f32)
    X = eye[None] + N
    Npow = N
    for _ in range(4):
        Npow = bmm(Npow, Npow, ((2,), (1,)))
        X = X + bmm(X, Npow, ((2,), (1,)))
    Linv = X * rinv[:, None, :]

    A = bmm(Linv, Linv, ((1,), (1,)))
    mu = mu_subj_ref[...]
    bvec = jnp.sum(A * mu[:, None, :], axis=2)
    cval = jnp.sum(mu * bvec, axis=1, keepdims=True)
    logdet = jnp.sum(jnp.log(d), axis=1, keepdims=True)
    Cval = (-0.5 * D * _LOG2PI) - logdet

    ahat_scr[:, 0:D, 0:D] = A
    ahat_scr[:, D:DA, 0:D] = -bvec[:, None, :]
    ahat_scr[:, 0:D, D:DA] = -bvec[:, :, None]
    ahat_scr[:, D:DA, D:DA] = (cval - 2.0 * Cval)[:, :, None]

    for b in range(B):
        XG = xgT_ref[b]
        xT = XG[0:DA]
        gT = XG[DA:DA + K]
        Wg = gT[:, None, :] * xT[None]
        S = bmm(Wg, jnp.broadcast_to(xT[None], (K, DA, T)), ((2,), (2,)))
        smom_scr[b * K:(b + 1) * K] = S

    Ah_sel = jnp.concatenate(
        [ahat_scr[pl.ds(sid_ref[b] * K, K)] for b in range(B)], axis=0)
    acc = jnp.sum(Ah_sel * smom_scr[...])
    nll = 0.5 * acc / float(B * T)

    ids_v = ids_ref[...]
    pio = jax.lax.broadcasted_iota(jnp.int32, (P, B), 0)
    pres = jnp.max((pio == ids_v).astype(f32), axis=1, keepdims=True)
    S_cnt = jnp.sum(pres)

    md = mu.reshape(P, K, D) - mu_pop_ref[...][None]
    msq = jnp.sum(jnp.sum(md * md, axis=2), axis=1, keepdims=True)
    mu_reg = jnp.sum(pres * msq)

    Ld = L.reshape(P, K, D, D) - Lp_ref[...][None]
    lsq = jnp.sum(jnp.sum(jnp.sum(Ld * Ld, axis=3), axis=2), axis=1,
                  keepdims=True)
    L_reg = jnp.sum(pres * lsq)

    reg = (S_cnt / _N_SUBJECTS) * (
        _LAMBDA_MU / 2.0 * mu_reg + _LAMBDA_L / 2.0 * L_reg)
    out_ref[0, 0] = nll + reg


def kernel(x, mu_pop, L_pop, mu_subj, L_subj, gamma, subject_ids):
    B, T, D = x.shape
    K = mu_pop.shape[0]
    P = mu_subj.shape[0]
    f32 = jnp.float32

    sid = subject_ids.astype(jnp.int32)
    xg = jnp.concatenate([x, jnp.ones((B, T, 1), f32), gamma], axis=2)
    xgT = jnp.swapaxes(xg, 1, 2)
    Ls = L_subj.reshape(P * K, D, D)
    mu_s = mu_subj.reshape(P * K, D)
    ids2 = sid.reshape(1, B)

    out = pl.pallas_call(
        _body,
        out_shape=jax.ShapeDtypeStruct((1, 1), f32),
        in_specs=[
            pl.BlockSpec(memory_space=pltpu.SMEM),
            pl.BlockSpec(memory_space=pltpu.VMEM),
            pl.BlockSpec(memory_space=pltpu.VMEM),
            pl.BlockSpec(memory_space=pltpu.VMEM),
            pl.BlockSpec(memory_space=pltpu.VMEM),
            pl.BlockSpec(memory_space=pltpu.VMEM),
            pl.BlockSpec(memory_space=pltpu.VMEM),
        ],
        out_specs=pl.BlockSpec(memory_space=pltpu.SMEM),
        scratch_shapes=[
            pltpu.VMEM((P * K, D + 1, D + 1), f32),
            pltpu.VMEM((B * K, D + 1, D + 1), f32),
        ],
    )(sid, ids2, xgT, mu_s, mu_pop, Ls, L_pop)
    return out[0, 0]


# speedup vs baseline: 1.8688x; 1.0378x over previous
import jax
import jax.numpy as jnp
import numpy as np
from jax.experimental import pallas as pl
from jax.experimental.pallas import tpu as pltpu

_LAMBDA_MU = 0.1
_LAMBDA_L = 0.1
_N_SUBJECTS = 16
_LOG2PI = float(np.log(2.0 * np.pi))


def _body(sid_ref, ids_ref, xgT_hbm, mu_subj_ref,
          mu_pop_ref, Ls_ref, Lp_ref, out_ref, ahat_scr, smom_scr,
          xgT_ref, dma_sem):
    B, T, D, K, P = 16, 512, 32, 8, 16
    DA = D + 1
    f32 = jnp.float32

    xgT_copy = pltpu.make_async_copy(xgT_hbm, xgT_ref, dma_sem)
    xgT_copy.start()

    L = Ls_ref[...]
    ii = jax.lax.broadcasted_iota(jnp.int32, (D, D), 0)
    jj = jax.lax.broadcasted_iota(jnp.int32, (D, D), 1)
    eye = (ii == jj).astype(f32)
    strict_lo = (ii > jj).astype(f32)

    d = jnp.sum(L * eye, axis=1)
    rinv = 1.0 / d
    N = -(L * strict_lo) * rinv[:, :, None]
    bmm = lambda a, b, dn: jax.lax.dot_general(
        a, b, (dn, ((0,), (0,))), preferred_element_type=f32)
    X = eye[None] + N
    Npow = N
    for _ in range(4):
        Npow = bmm(Npow, Npow, ((2,), (1,)))
        X = X + bmm(X, Npow, ((2,), (1,)))
    Linv = X * rinv[:, None, :]

    A = bmm(Linv, Linv, ((1,), (1,)))
    mu = mu_subj_ref[...]
    bvec = jnp.sum(A * mu[:, None, :], axis=2)
    cval = jnp.sum(mu * bvec, axis=1, keepdims=True)
    logdet = jnp.sum(jnp.log(d), axis=1, keepdims=True)
    Cval = (-0.5 * D * _LOG2PI) - logdet

    ahat_scr[:, 0:D, 0:D] = A
    ahat_scr[:, D:DA, 0:D] = -bvec[:, None, :]
    ahat_scr[:, 0:D, D:DA] = -bvec[:, :, None]
    ahat_scr[:, D:DA, D:DA] = (cval - 2.0 * Cval)[:, :, None]

    xgT_copy.wait()
    for b in range(B):
        XG = xgT_ref[b]
        xT = XG[0:DA]
        gT = XG[DA:DA + K]
        Wg = gT[:, None, :] * xT[None]
        S = bmm(Wg, jnp.broadcast_to(xT[None], (K, DA, T)), ((2,), (2,)))
        smom_scr[b * K:(b + 1) * K] = S

    Ah_sel = jnp.concatenate(
        [ahat_scr[pl.ds(sid_ref[b] * K, K)] for b in range(B)], axis=0)
    acc = jnp.sum(Ah_sel * smom_scr[...])
    nll = 0.5 * acc / float(B * T)

    ids_v = ids_ref[...]
    pio = jax.lax.broadcasted_iota(jnp.int32, (P, B), 0)
    pres = jnp.max((pio == ids_v).astype(f32), axis=1, keepdims=True)
    S_cnt = jnp.sum(pres)

    md = mu.reshape(P, K, D) - mu_pop_ref[...][None]
    msq = jnp.sum(jnp.sum(md * md, axis=2), axis=1, keepdims=True)
    mu_reg = jnp.sum(pres * msq)

    Ld = L.reshape(P, K, D, D) - Lp_ref[...][None]
    lsq = jnp.sum(jnp.sum(jnp.sum(Ld * Ld, axis=3), axis=2), axis=1,
                  keepdims=True)
    L_reg = jnp.sum(pres * lsq)

    reg = (S_cnt / _N_SUBJECTS) * (
        _LAMBDA_MU / 2.0 * mu_reg + _LAMBDA_L / 2.0 * L_reg)
    out_ref[0, 0] = nll + reg


def kernel(x, mu_pop, L_pop, mu_subj, L_subj, gamma, subject_ids):
    B, T, D = x.shape
    K = mu_pop.shape[0]
    P = mu_subj.shape[0]
    f32 = jnp.float32

    sid = subject_ids.astype(jnp.int32)
    xg = jnp.concatenate([x, jnp.ones((B, T, 1), f32), gamma], axis=2)
    xgT = jnp.swapaxes(xg, 1, 2)
    Ls = L_subj.reshape(P * K, D, D)
    mu_s = mu_subj.reshape(P * K, D)
    ids2 = sid.reshape(1, B)

    out = pl.pallas_call(
        _body,
        out_shape=jax.ShapeDtypeStruct((1, 1), f32),
        in_specs=[
            pl.BlockSpec(memory_space=pltpu.SMEM),
            pl.BlockSpec(memory_space=pltpu.VMEM),
            pl.BlockSpec(memory_space=pl.ANY),
            pl.BlockSpec(memory_space=pltpu.VMEM),
            pl.BlockSpec(memory_space=pltpu.VMEM),
            pl.BlockSpec(memory_space=pltpu.VMEM),
            pl.BlockSpec(memory_space=pltpu.VMEM),
        ],
        out_specs=pl.BlockSpec(memory_space=pltpu.SMEM),
        scratch_shapes=[
            pltpu.VMEM((P * K, D + 1, D + 1), f32),
            pltpu.VMEM((B * K, D + 1, D + 1), f32),
            pltpu.VMEM((B, D + 1 + K, T), f32),
            pltpu.SemaphoreType.DMA,
        ],
    )(sid, ids2, xgT, mu_s, mu_pop, Ls, L_pop)
    return out[0, 0]
